# baseline (speedup 1.0000x reference)
import functools

import jax
import jax.numpy as jnp
from jax import lax
from jax.experimental import pallas as pl
from jax.experimental.pallas import tpu as pltpu
from jax.experimental.pallas import tpu_sc as plsc

BATCH = 16384
EMB_DIM = 32
NUM_WORKERS = 32
B_PER_W = BATCH // NUM_WORKERS
CB = 128
N_CH = B_PER_W // CB
LANES = 16
EB = 32768


def _relayout_body(in_ref, out_ref):
  x = in_ref[...]
  eye = (lax.broadcasted_iota(jnp.int32, (EMB_DIM, EMB_DIM), 0)
         == lax.broadcasted_iota(jnp.int32, (EMB_DIM, EMB_DIM), 1)
         ).astype(jnp.float32)
  y = lax.dot_general(x, eye, (((0,), (0,)), ((), ())),
                      preferred_element_type=jnp.float32)
  q = EB // 4
  out_ref[...] = jnp.concatenate(
      [y[q * a:q * (a + 1), :] for a in range(4)], axis=1)


def _relayout(tT):
  n = tT.shape[1]
  grid = (n + EB - 1) // EB
  return pl.pallas_call(
      _relayout_body,
      grid=(grid,),
      in_specs=[pl.BlockSpec((EMB_DIM, EB), lambda i: (0, i))],
      out_specs=pl.BlockSpec((EB // 4, 4 * EMB_DIM), lambda i: (i, 0)),
      out_shape=jax.ShapeDtypeStruct((grid * (EB // 4), 4 * EMB_DIM),
                                     jnp.float32),
  )(tT)


def _fire(c, eh2, et2, rf2, ri2, h_sr, r_sr, t_sr, bufs, sem):
  h1, t1, h2, t2, r1, r2 = bufs
  hi = h_sr.at[c]
  ri_ = r_sr.at[c]
  ti = t_sr.at[c]
  return [
      pltpu.async_copy(eh2.at[hi], h1, sem),
      pltpu.async_copy(et2.at[ti], t1, sem),
      pltpu.async_copy(et2.at[hi], h2, sem),
      pltpu.async_copy(eh2.at[ti], t2, sem),
      pltpu.async_copy(rf2.at[ri_], r1, sem),
      pltpu.async_copy(ri2.at[ri_], r2, sem),
  ]


def _simple_body(heads_hbm, rels_hbm, tails_hbm, eh2, et2, rf2, ri2,
                 out_hbm,
                 h_idx, r_idx, t_idx, h_sr, r_sr, t_sr,
                 h1, t1, h2, t2, r1, r2,
                 out_v, sem):
  wid = lax.axis_index("s") * 2 + lax.axis_index("c")
  base_tile = wid * 4

  pltpu.sync_copy(heads_hbm.at[pl.ds(base_tile, 4)], h_idx)
  pltpu.sync_copy(rels_hbm.at[pl.ds(base_tile, 4)], r_idx)
  pltpu.sync_copy(tails_hbm.at[pl.ds(base_tile, 4)], t_idx)

  def _sr(v):
    return lax.shift_left(lax.shift_right_logical(v, 15), 13) | (v & 8191)

  for j in range(4):
    for v in range(8):
      s = pl.ds(v * LANES, LANES)
      h_sr[j, s] = _sr(h_idx[j, s])
      r_sr[j, s] = _sr(r_idx[j, s])
      t_sr[j, s] = _sr(t_idx[j, s])

  bufs = (h1, t1, h2, t2, r1, r2)
  lane = lax.iota(jnp.int32, LANES)

  for c in range(N_CH):
    pend = _fire(c, eh2, et2, rf2, ri2, h_sr, r_sr, t_sr, bufs, sem)
    for cp in pend:
      cp.wait()

    def group(i, carry, c=c):
      acc = jnp.zeros((LANES,), jnp.float32)
      flat0 = c * CB + i * LANES
      j = flat0 // 128
      col0 = lax.rem(flat0, 128)
      hov = (lax.shift_right_logical(h_idx[j, pl.ds(col0, LANES)], 13) & 3) * EMB_DIM
      rov = (lax.shift_right_logical(r_idx[j, pl.ds(col0, LANES)], 13) & 3) * EMB_DIM
      tov = (lax.shift_right_logical(t_idx[j, pl.ds(col0, LANES)], 13) & 3) * EMB_DIM
      for k in range(LANES):
        row = i * LANES + k
        ho = hov[k]
        ro = rov[k]
        to = tov[k]
        a0 = (h1[row, pl.ds(ho, LANES)]
              * r1[row, pl.ds(ro, LANES)]
              * t1[row, pl.ds(to, LANES)]
              + h2[row, pl.ds(ho, LANES)]
              * r2[row, pl.ds(ro, LANES)]
              * t2[row, pl.ds(to, LANES)])
        a1 = (h1[row, pl.ds(ho + LANES, LANES)]
              * r1[row, pl.ds(ro + LANES, LANES)]
              * t1[row, pl.ds(to + LANES, LANES)]
              + h2[row, pl.ds(ho + LANES, LANES)]
              * r2[row, pl.ds(ro + LANES, LANES)]
              * t2[row, pl.ds(to + LANES, LANES)])
        acc = jnp.where(lane == k, jnp.sum(a0 + a1), acc)
      out_v[pl.ds(c * CB + i * LANES, LANES)] = acc * 0.5
      return carry

    lax.fori_loop(0, CB // LANES, group, 0)

  pltpu.sync_copy(out_v, out_hbm.at[pl.ds(wid * B_PER_W, B_PER_W)])


@jax.jit
def _simple_sc(heads, rels, tails, eh, et, rf, ri):
  mesh = plsc.VectorSubcoreMesh(core_axis_name="c", subcore_axis_name="s")
  run = pl.kernel(
      _simple_body,
      out_type=jax.ShapeDtypeStruct((BATCH,), jnp.float32),
      mesh=mesh,
      compiler_params=pltpu.CompilerParams(
          needs_layout_passes=False, use_tc_tiling_on_sc=True),
      scratch_types=[
          pltpu.VMEM((4, 128), jnp.int32),
          pltpu.VMEM((4, 128), jnp.int32),
          pltpu.VMEM((4, 128), jnp.int32),
          pltpu.VMEM((4, 128), jnp.int32),
          pltpu.VMEM((4, 128), jnp.int32),
          pltpu.VMEM((4, 128), jnp.int32),
          pltpu.VMEM((CB, 128), jnp.float32),
          pltpu.VMEM((CB, 128), jnp.float32),
          pltpu.VMEM((CB, 128), jnp.float32),
          pltpu.VMEM((CB, 128), jnp.float32),
          pltpu.VMEM((CB, 128), jnp.float32),
          pltpu.VMEM((CB, 128), jnp.float32),
          pltpu.VMEM((B_PER_W,), jnp.float32),
          pltpu.SemaphoreType.DMA,
      ],
  )
  heads2 = heads.astype(jnp.int32).reshape(128, 128)
  rels2 = rels.astype(jnp.int32).reshape(128, 128)
  tails2 = tails.astype(jnp.int32).reshape(128, 128)
  eh2 = _relayout(eh.T)
  et2 = _relayout(et.T)
  rf2 = _relayout(rf.T)
  ri2 = _relayout(ri.T)
  return run(heads2, rels2, tails2, eh2, et2, rf2, ri2)


def kernel(heads, rels, tails, ent_embeds_head, ent_embeds_tail,
           rel_embeds_for, rel_embeds_inv):
  return _simple_sc(heads, rels, tails, ent_embeds_head, ent_embeds_tail,
                    rel_embeds_for, rel_embeds_inv)


# speedup vs baseline: 1.5521x; 1.5521x over previous
import functools

import jax
import jax.numpy as jnp
from jax import lax
from jax.experimental import pallas as pl
from jax.experimental.pallas import tpu as pltpu
from jax.experimental.pallas import tpu_sc as plsc

BATCH = 16384
EMB_DIM = 32
NUM_WORKERS = 32
B_PER_W = BATCH // NUM_WORKERS
CB = 128
N_CH = B_PER_W // CB
LANES = 16
EB = 32768


def _relayout_body(in_ref, out_ref):
  x = in_ref[...]
  q = EB // 4
  row = lax.broadcasted_iota(jnp.int32, (EMB_DIM, 4 * EMB_DIM), 0)
  col = lax.broadcasted_iota(jnp.int32, (EMB_DIM, 4 * EMB_DIM), 1)
  acc = None
  for a in range(4):
    ea = (col == row + a * EMB_DIM).astype(jnp.float32)
    za = lax.dot_general(x[:, q * a:q * (a + 1)], ea,
                         (((0,), (0,)), ((), ())),
                         preferred_element_type=jnp.float32)
    acc = za if acc is None else acc + za
  out_ref[...] = acc


def _relayout(tT):
  n = tT.shape[1]
  grid = (n + EB - 1) // EB
  return pl.pallas_call(
      _relayout_body,
      grid=(grid,),
      in_specs=[pl.BlockSpec((EMB_DIM, EB), lambda i: (0, i))],
      out_specs=pl.BlockSpec((EB // 4, 4 * EMB_DIM), lambda i: (i, 0)),
      out_shape=jax.ShapeDtypeStruct((grid * (EB // 4), 4 * EMB_DIM),
                                     jnp.float32),
  )(tT)


def _fire(c, eh2, et2, rf2, ri2, h_sr, r_sr, t_sr, bufs, sem):
  h1, t1, h2, t2, r1, r2 = bufs
  hi = h_sr.at[c]
  ri_ = r_sr.at[c]
  ti = t_sr.at[c]
  return [
      pltpu.async_copy(eh2.at[hi], h1, sem),
      pltpu.async_copy(et2.at[ti], t1, sem),
      pltpu.async_copy(et2.at[hi], h2, sem),
      pltpu.async_copy(eh2.at[ti], t2, sem),
      pltpu.async_copy(rf2.at[ri_], r1, sem),
      pltpu.async_copy(ri2.at[ri_], r2, sem),
  ]


def _simple_body(heads_hbm, rels_hbm, tails_hbm, eh2, et2, rf2, ri2,
                 out_hbm,
                 h_idx, r_idx, t_idx, h_sr, r_sr, t_sr,
                 h1, t1, h2, t2, r1, r2,
                 out_v, sem):
  wid = lax.axis_index("s") * 2 + lax.axis_index("c")
  base_tile = wid * 4

  pltpu.sync_copy(heads_hbm.at[pl.ds(base_tile, 4)], h_idx)
  pltpu.sync_copy(rels_hbm.at[pl.ds(base_tile, 4)], r_idx)
  pltpu.sync_copy(tails_hbm.at[pl.ds(base_tile, 4)], t_idx)

  def _sr(v):
    return lax.shift_left(lax.shift_right_logical(v, 15), 13) | (v & 8191)

  for j in range(4):
    for v in range(8):
      s = pl.ds(v * LANES, LANES)
      h_sr[j, s] = _sr(h_idx[j, s])
      r_sr[j, s] = _sr(r_idx[j, s])
      t_sr[j, s] = _sr(t_idx[j, s])

  bufs = (h1, t1, h2, t2, r1, r2)
  lane = lax.iota(jnp.int32, LANES)

  for c in range(N_CH):
    pend = _fire(c, eh2, et2, rf2, ri2, h_sr, r_sr, t_sr, bufs, sem)
    for cp in pend:
      cp.wait()

    def group(i, carry, c=c):
      acc = jnp.zeros((LANES,), jnp.float32)
      flat0 = c * CB + i * LANES
      j = flat0 // 128
      col0 = lax.rem(flat0, 128)
      hov = (lax.shift_right_logical(h_idx[j, pl.ds(col0, LANES)], 13) & 3) * EMB_DIM
      rov = (lax.shift_right_logical(r_idx[j, pl.ds(col0, LANES)], 13) & 3) * EMB_DIM
      tov = (lax.shift_right_logical(t_idx[j, pl.ds(col0, LANES)], 13) & 3) * EMB_DIM
      for k in range(LANES):
        row = i * LANES + k
        ho = hov[k]
        ro = rov[k]
        to = tov[k]
        a0 = (h1[row, pl.ds(ho, LANES)]
              * r1[row, pl.ds(ro, LANES)]
              * t1[row, pl.ds(to, LANES)]
              + h2[row, pl.ds(ho, LANES)]
              * r2[row, pl.ds(ro, LANES)]
              * t2[row, pl.ds(to, LANES)])
        a1 = (h1[row, pl.ds(ho + LANES, LANES)]
              * r1[row, pl.ds(ro + LANES, LANES)]
              * t1[row, pl.ds(to + LANES, LANES)]
              + h2[row, pl.ds(ho + LANES, LANES)]
              * r2[row, pl.ds(ro + LANES, LANES)]
              * t2[row, pl.ds(to + LANES, LANES)])
        acc = jnp.where(lane == k, jnp.sum(a0 + a1), acc)
      out_v[pl.ds(c * CB + i * LANES, LANES)] = acc * 0.5
      return carry

    lax.fori_loop(0, CB // LANES, group, 0)

  pltpu.sync_copy(out_v, out_hbm.at[pl.ds(wid * B_PER_W, B_PER_W)])


@jax.jit
def _simple_sc(heads, rels, tails, eh, et, rf, ri):
  mesh = plsc.VectorSubcoreMesh(core_axis_name="c", subcore_axis_name="s")
  run = pl.kernel(
      _simple_body,
      out_type=jax.ShapeDtypeStruct((BATCH,), jnp.float32),
      mesh=mesh,
      compiler_params=pltpu.CompilerParams(
          needs_layout_passes=False, use_tc_tiling_on_sc=True),
      scratch_types=[
          pltpu.VMEM((4, 128), jnp.int32),
          pltpu.VMEM((4, 128), jnp.int32),
          pltpu.VMEM((4, 128), jnp.int32),
          pltpu.VMEM((4, 128), jnp.int32),
          pltpu.VMEM((4, 128), jnp.int32),
          pltpu.VMEM((4, 128), jnp.int32),
          pltpu.VMEM((CB, 128), jnp.float32),
          pltpu.VMEM((CB, 128), jnp.float32),
          pltpu.VMEM((CB, 128), jnp.float32),
          pltpu.VMEM((CB, 128), jnp.float32),
          pltpu.VMEM((CB, 128), jnp.float32),
          pltpu.VMEM((CB, 128), jnp.float32),
          pltpu.VMEM((B_PER_W,), jnp.float32),
          pltpu.SemaphoreType.DMA,
      ],
  )
  heads2 = heads.astype(jnp.int32).reshape(128, 128)
  rels2 = rels.astype(jnp.int32).reshape(128, 128)
  tails2 = tails.astype(jnp.int32).reshape(128, 128)
  eh2 = _relayout(eh.T)
  et2 = _relayout(et.T)
  rf2 = _relayout(rf.T)
  ri2 = _relayout(ri.T)
  return run(heads2, rels2, tails2, eh2, et2, rf2, ri2)


def kernel(heads, rels, tails, ent_embeds_head, ent_embeds_tail,
           rel_embeds_for, rel_embeds_inv):
  return _simple_sc(heads, rels, tails, ent_embeds_head, ent_embeds_tail,
                    rel_embeds_for, rel_embeds_inv)


# speedup vs baseline: 1.5539x; 1.0012x over previous
import jax
import jax.numpy as jnp
from jax import lax
from jax.experimental import pallas as pl
from jax.experimental.pallas import tpu as pltpu
from jax.experimental.pallas import tpu_sc as plsc

BATCH = 16384
EMB_DIM = 32
NUM_WORKERS = 32
B_PER_W = BATCH // NUM_WORKERS
CB = 128
N_CH = B_PER_W // CB
LANES = 16
EB = 32768


def _relayout_body(in_ref, out_ref):
  x = in_ref[...]
  q = EB // 4
  row = lax.broadcasted_iota(jnp.int32, (EMB_DIM, 4 * EMB_DIM), 0)
  col = lax.broadcasted_iota(jnp.int32, (EMB_DIM, 4 * EMB_DIM), 1)
  acc = None
  for a in range(4):
    ea = (col == row + a * EMB_DIM).astype(jnp.float32)
    za = lax.dot_general(x[:, q * a:q * (a + 1)], ea,
                         (((0,), (0,)), ((), ())),
                         preferred_element_type=jnp.float32)
    acc = za if acc is None else acc + za
  out_ref[...] = acc


def _relayout(tT):
  n = tT.shape[1]
  grid = (n + EB - 1) // EB
  return pl.pallas_call(
      _relayout_body,
      grid=(grid,),
      in_specs=[pl.BlockSpec((EMB_DIM, EB), lambda i: (0, i))],
      out_specs=pl.BlockSpec((EB // 4, 4 * EMB_DIM), lambda i: (i, 0)),
      out_shape=jax.ShapeDtypeStruct((grid * (EB // 4), 4 * EMB_DIM),
                                     jnp.float32),
  )(tT)


def _fire(c, eh2, et2, rf2, ri2, h_sr, r_sr, t_sr, bufs, sem):
  h1, t1, h2, t2, r1, r2 = bufs
  hi = h_sr.at[c]
  ri_ = r_sr.at[c]
  ti = t_sr.at[c]
  return [
      pltpu.async_copy(eh2.at[hi], h1, sem),
      pltpu.async_copy(et2.at[ti], t1, sem),
      pltpu.async_copy(et2.at[hi], h2, sem),
      pltpu.async_copy(eh2.at[ti], t2, sem),
      pltpu.async_copy(rf2.at[ri_], r1, sem),
      pltpu.async_copy(ri2.at[ri_], r2, sem),
  ]


def _simple_body(heads_hbm, rels_hbm, tails_hbm, eh2, et2, rf2, ri2,
                 out_hbm,
                 h_idx, r_idx, t_idx, h_sr, r_sr, t_sr,
                 h1, t1, h2, t2, r1, r2,
                 out_v, sem):
  wid = lax.axis_index("s") * 2 + lax.axis_index("c")
  base_tile = wid * 4

  pltpu.sync_copy(heads_hbm.at[pl.ds(base_tile, 4)], h_idx)
  pltpu.sync_copy(rels_hbm.at[pl.ds(base_tile, 4)], r_idx)
  pltpu.sync_copy(tails_hbm.at[pl.ds(base_tile, 4)], t_idx)

  def _sr(v):
    return lax.shift_left(lax.shift_right_logical(v, 15), 13) | (v & 8191)

  for j in range(4):
    for v in range(8):
      s = pl.ds(v * LANES, LANES)
      h_sr[j, s] = _sr(h_idx[j, s])
      r_sr[j, s] = _sr(r_idx[j, s])
      t_sr[j, s] = _sr(t_idx[j, s])

  bufs = (h1, t1, h2, t2, r1, r2)
  lane = lax.iota(jnp.int32, LANES)

  for c in range(N_CH):
    pend = _fire(c, eh2, et2, rf2, ri2, h_sr, r_sr, t_sr, bufs, sem)
    for cp in pend:
      cp.wait()

    def group(i, carry, c=c):
      acc = jnp.zeros((LANES,), jnp.float32)
      flat0 = c * CB + i * LANES
      j = flat0 // 128
      col0 = lax.rem(flat0, 128)
      hov = (lax.shift_right_logical(h_idx[j, pl.ds(col0, LANES)], 13) & 3) * EMB_DIM
      rov = (lax.shift_right_logical(r_idx[j, pl.ds(col0, LANES)], 13) & 3) * EMB_DIM
      tov = (lax.shift_right_logical(t_idx[j, pl.ds(col0, LANES)], 13) & 3) * EMB_DIM
      for k in range(LANES):
        row = i * LANES + k
        ho = hov[k]
        ro = rov[k]
        to = tov[k]
        a0 = (h1[row, pl.ds(ho, LANES)]
              * r1[row, pl.ds(ro, LANES)]
              * t1[row, pl.ds(to, LANES)]
              + h2[row, pl.ds(ho, LANES)]
              * r2[row, pl.ds(ro, LANES)]
              * t2[row, pl.ds(to, LANES)])
        a1 = (h1[row, pl.ds(ho + LANES, LANES)]
              * r1[row, pl.ds(ro + LANES, LANES)]
              * t1[row, pl.ds(to + LANES, LANES)]
              + h2[row, pl.ds(ho + LANES, LANES)]
              * r2[row, pl.ds(ro + LANES, LANES)]
              * t2[row, pl.ds(to + LANES, LANES)])
        acc = jnp.where(lane == k, jnp.sum(a0 + a1), acc)
      out_v[pl.ds(c * CB + i * LANES, LANES)] = acc * 0.5
      return carry

    lax.fori_loop(0, CB // LANES, group, 0)

  pltpu.sync_copy(out_v, out_hbm.at[pl.ds(wid * B_PER_W, B_PER_W)])


@jax.jit
def _simple_sc(heads, rels, tails, eh, et, rf, ri):
  mesh = plsc.VectorSubcoreMesh(core_axis_name="c", subcore_axis_name="s")
  run = pl.kernel(
      _simple_body,
      out_type=jax.ShapeDtypeStruct((BATCH,), jnp.float32),
      mesh=mesh,
      compiler_params=pltpu.CompilerParams(
          needs_layout_passes=False, use_tc_tiling_on_sc=True),
      scratch_types=[
          pltpu.VMEM((4, 128), jnp.int32),
          pltpu.VMEM((4, 128), jnp.int32),
          pltpu.VMEM((4, 128), jnp.int32),
          pltpu.VMEM((4, 128), jnp.int32),
          pltpu.VMEM((4, 128), jnp.int32),
          pltpu.VMEM((4, 128), jnp.int32),
          pltpu.VMEM((CB, 128), jnp.float32),
          pltpu.VMEM((CB, 128), jnp.float32),
          pltpu.VMEM((CB, 128), jnp.float32),
          pltpu.VMEM((CB, 128), jnp.float32),
          pltpu.VMEM((CB, 128), jnp.float32),
          pltpu.VMEM((CB, 128), jnp.float32),
          pltpu.VMEM((B_PER_W,), jnp.float32),
          pltpu.SemaphoreType.DMA,
      ],
  )
  heads2 = heads.astype(jnp.int32).reshape(128, 128)
  rels2 = rels.astype(jnp.int32).reshape(128, 128)
  tails2 = tails.astype(jnp.int32).reshape(128, 128)
  eh2 = _relayout(eh.T)
  et2 = _relayout(et.T)
  rf2 = _relayout(rf.T)
  ri2 = _relayout(ri.T)
  return run(heads2, rels2, tails2, eh2, et2, rf2, ri2)


def kernel(heads, rels, tails, ent_embeds_head, ent_embeds_tail,
           rel_embeds_for, rel_embeds_inv):
  return _simple_sc(heads, rels, tails, ent_embeds_head, ent_embeds_tail,
                    rel_embeds_for, rel_embeds_inv)
